# Initial kernel scaffold; baseline (speedup 1.0000x reference)
import jax
import jax.numpy as jnp
from jax.experimental import pallas as pl


def kernel(x, W1, b1, W2, b2, eps, edge_index, pos_edge_index, neg_edge_index):
    raise NotImplementedError("write your pallas kernel here")


# speedup vs baseline: 4.5969x; 4.5969x over previous
import functools

import jax
import jax.numpy as jnp
from jax import lax
from jax.experimental import pallas as pl
from jax.experimental.pallas import tpu as pltpu
from jax.experimental.pallas import tpu_sc as plsc

N = 10000
E = 320000
D_IN = 128
EMB = 32

NC = 2
NS = 16
N_PAD = 10240
STRIPE = N_PAD // NS
CH = 80
ER = E // CH

_MESH = plsc.VectorSubcoreMesh(
    core_axis_name="c", subcore_axis_name="s", num_cores=NC, num_subcores=NS)
_SC_PARAMS = pltpu.CompilerParams(
    use_tc_tiling_on_sc=False, needs_layout_passes=False)


_DEG_RPT = ER // NS


@functools.partial(
    pl.kernel,
    out_type=jax.ShapeDtypeStruct((2, N_PAD, 8), jnp.float32),
    mesh=_MESH,
    compiler_params=_SC_PARAMS,
    scratch_types=[
        pltpu.VMEM((_DEG_RPT, CH), jnp.int32),
        pltpu.VMEM((CH, 8), jnp.float32),
        pltpu.VMEM((STRIPE, 8), jnp.float32),
        pltpu.VMEM_SHARED((N_PAD, 8), jnp.float32),
    ],
)
def _deg_kernel(edges_hbm, ones_hbm, zeros_hbm, out_hbm,
                idx_v, ones_v, zrow_v, acc_sh):
    c = lax.axis_index("c")
    s = lax.axis_index("s")
    pltpu.sync_copy(zeros_hbm, zrow_v)
    pltpu.sync_copy(zrow_v, acc_sh.at[pl.ds(s * STRIPE, STRIPE)])
    pltpu.sync_copy(ones_hbm, ones_v)
    plsc.subcore_barrier()
    pltpu.sync_copy(edges_hbm.at[c, s], idx_v)

    def body(j, carry):
        pltpu.sync_copy(ones_v, acc_sh.at[idx_v.at[j]], add=True)
        return carry

    lax.fori_loop(0, _DEG_RPT, body, 0)
    plsc.subcore_barrier()
    pltpu.sync_copy(acc_sh.at[pl.ds(s * STRIPE, STRIPE)],
                    out_hbm.at[c, pl.ds(s * STRIPE, STRIPE)])


_CONV_RPT = (ER // NC) // NS


def _make_conv(F):
    @functools.partial(
        pl.kernel,
        out_type=jax.ShapeDtypeStruct((2, N_PAD, F), jnp.float32),
        mesh=_MESH,
        compiler_params=_SC_PARAMS,
        scratch_types=[
            pltpu.VMEM((_CONV_RPT, CH), jnp.int32),
            pltpu.VMEM((_CONV_RPT, CH), jnp.int32),
            pltpu.VMEM((CH, F), jnp.float32),
            pltpu.VMEM((STRIPE, F), jnp.float32),
            pltpu.VMEM_SHARED((N_PAD, F), jnp.float32),
            pltpu.SemaphoreType.DMA,
        ],
    )
    def conv(feat_hbm, edges_hbm, zeros_hbm, out_hbm,
             src_v, dst_v, rows_v, zrow_v, acc_sh, sem):
        c = lax.axis_index("c")
        s = lax.axis_index("s")
        pltpu.sync_copy(zeros_hbm, zrow_v)
        pltpu.sync_copy(zrow_v, acc_sh.at[pl.ds(s * STRIPE, STRIPE)])
        plsc.subcore_barrier()
        w = c * NS + s
        pltpu.sync_copy(edges_hbm.at[0, w], src_v)
        pltpu.sync_copy(edges_hbm.at[1, w], dst_v)

        def body(j, carry):
            pltpu.async_copy(feat_hbm.at[src_v.at[j]], rows_v, sem).wait()
            pltpu.sync_copy(rows_v, acc_sh.at[dst_v.at[j]], add=True)
            return carry

        lax.fori_loop(0, _CONV_RPT, body, 0)
        plsc.subcore_barrier()
        pltpu.sync_copy(acc_sh.at[pl.ds(s * STRIPE, STRIPE)],
                        out_hbm.at[c, pl.ds(s * STRIPE, STRIPE)])

    return conv


_conv32 = _make_conv(EMB)
_conv64 = _make_conv(2 * EMB)

_DEC_RPT = ER // NS
_DEC_EPT = _DEC_RPT * CH


@functools.partial(
    pl.kernel,
    out_type=jax.ShapeDtypeStruct((2, NS, _DEC_RPT, CH), jnp.float32),
    mesh=_MESH,
    compiler_params=_SC_PARAMS,
    scratch_types=[
        pltpu.VMEM((_DEC_RPT, CH), jnp.int32),
        pltpu.VMEM((_DEC_RPT, CH), jnp.int32),
        pltpu.VMEM((CH, EMB), jnp.float32),
        pltpu.VMEM((CH, EMB), jnp.float32),
        pltpu.VMEM((_DEC_RPT, CH), jnp.float32),
        pltpu.SemaphoreType.DMA,
        pltpu.SemaphoreType.DMA,
    ],
)
def _dec_kernel(z_hbm, dec_hbm, out_hbm,
                u_v, v_v, zu_v, zv_v, res_v, sem_u, sem_v):
    c = lax.axis_index("c")
    s = lax.axis_index("s")
    pltpu.sync_copy(dec_hbm.at[c, 0, s], u_v)
    pltpu.sync_copy(dec_hbm.at[c, 1, s], v_v)
    lanes = lax.iota(jnp.int32, 16)

    def body(j, carry):
        cu = pltpu.async_copy(z_hbm.at[u_v.at[j]], zu_v, sem_u)
        cv = pltpu.async_copy(z_hbm.at[v_v.at[j]], zv_v, sem_v)
        cu.wait()
        cv.wait()
        for g in range(CH // 16):
            rows = lanes + (g * 16)
            acc = jnp.zeros((16,), jnp.float32)
            for k in range(EMB):
                kk = jnp.full((16,), k, jnp.int32)
                acc = acc + (plsc.load_gather(zu_v, [rows, kk]) *
                             plsc.load_gather(zv_v, [rows, kk]))
            res_v[j, pl.ds(g * 16, 16)] = acc
        return carry

    lax.fori_loop(0, _DEC_RPT, body, 0)
    pltpu.sync_copy(res_v, out_hbm.at[c, s])


def _rs(deg2d):
    return lax.rsqrt(jnp.maximum(jnp.sum(deg2d, axis=-1, keepdims=True), 1.0))


def _enc1_body(x_ref, w_ref, d_ref, f1_ref):
    rs_out = _rs(d_ref[0])
    y1 = jnp.dot(x_ref[...], w_ref[...], preferred_element_type=jnp.float32,
                 precision=lax.Precision.HIGHEST)
    f1_ref[...] = y1 * rs_out


def _enc2_body(agg_ref, d_ref, w2_ref, b1_ref, f2_ref):
    agg = agg_ref[0] + agg_ref[1]
    h1 = jnp.maximum(agg * _rs(d_ref[1]) + b1_ref[...], 0.0)
    y2 = jnp.dot(h1, w2_ref[...], preferred_element_type=jnp.float32,
                 precision=lax.Precision.HIGHEST)
    f2_ref[...] = y2 * _rs(d_ref[0])


def _fin_body(agg_ref, d_ref, b2_ref, eps_ref, mu_ref, sig_ref, z_ref):
    agg = agg_ref[0] + agg_ref[1]
    h2 = agg * _rs(d_ref[1]) + b2_ref[...]
    mu = h2[:, :EMB]
    sig = jnp.exp(h2[:, EMB:] * 0.5)
    mu_ref[...] = mu
    sig_ref[...] = sig
    z_ref[...] = mu + sig * eps_ref[...]


_enc1 = pl.pallas_call(
    _enc1_body, out_shape=jax.ShapeDtypeStruct((N_PAD, EMB), jnp.float32))
_enc2 = pl.pallas_call(
    _enc2_body, out_shape=jax.ShapeDtypeStruct((N_PAD, 2 * EMB), jnp.float32))
_fin = pl.pallas_call(
    _fin_body, out_shape=[jax.ShapeDtypeStruct((N_PAD, EMB), jnp.float32),
                          jax.ShapeDtypeStruct((N_PAD, EMB), jnp.float32),
                          jax.ShapeDtypeStruct((N_PAD, EMB), jnp.float32)])


def kernel(x, W1, b1, W2, b2, eps, edge_index, pos_edge_index, neg_edge_index):
    x_p = jnp.pad(x, ((0, N_PAD - N), (0, 0)))
    eps_p = jnp.pad(eps, ((0, N_PAD - N), (0, 0)))
    edges_deg = edge_index.reshape(2, NS, _DEG_RPT, CH)
    edges_conv = edge_index.reshape(2, NC * NS, _CONV_RPT, CH)
    dec_r = jnp.stack([pos_edge_index.reshape(2, NS, _DEC_RPT, CH),
                       neg_edge_index.reshape(2, NS, _DEC_RPT, CH)])
    ones8 = jnp.zeros((CH, 8), jnp.float32).at[:, 0].set(1.0)
    z8 = jnp.zeros((STRIPE, 8), jnp.float32)
    z32 = jnp.zeros((STRIPE, EMB), jnp.float32)
    z64 = jnp.zeros((STRIPE, 2 * EMB), jnp.float32)

    deg = _deg_kernel(edges_deg, ones8, z8)
    f1 = _enc1(x_p, W1, deg)
    agg1 = _conv32(f1, edges_conv, z32)
    f2 = _enc2(agg1, deg, W2, b1.reshape(1, EMB))
    agg2 = _conv64(f2, edges_conv, z64)
    mu_p, sig_p, z_p = _fin(agg2, deg, b2.reshape(1, 2 * EMB), eps_p)
    scores = _dec_kernel(z_p, dec_r).reshape(2, E)
    return scores[0], scores[1], mu_p[:N], sig_p[:N]


# speedup vs baseline: 6.6353x; 1.4434x over previous
import functools

import jax
import jax.numpy as jnp
from jax import lax
from jax.experimental import pallas as pl
from jax.experimental.pallas import tpu as pltpu
from jax.experimental.pallas import tpu_sc as plsc

N = 10000
E = 320000
D_IN = 128
EMB = 32

NC = 2
NS = 16
N_PAD = 10240
STRIPE = N_PAD // NS
CH = 80
ER = E // CH

_MESH = plsc.VectorSubcoreMesh(
    core_axis_name="c", subcore_axis_name="s", num_cores=NC, num_subcores=NS)
_SC_PARAMS = pltpu.CompilerParams(
    use_tc_tiling_on_sc=False, needs_layout_passes=False)


_DEG_RPT = ER // NS


@functools.partial(
    pl.kernel,
    out_type=jax.ShapeDtypeStruct((2, N_PAD, 8), jnp.float32),
    mesh=_MESH,
    compiler_params=_SC_PARAMS,
    scratch_types=[
        pltpu.VMEM((_DEG_RPT, CH), jnp.int32),
        pltpu.VMEM((CH, 8), jnp.float32),
        pltpu.VMEM((STRIPE, 8), jnp.float32),
        pltpu.VMEM_SHARED((N_PAD, 8), jnp.float32),
    ],
)
def _deg_kernel(edges_hbm, ones_hbm, zeros_hbm, out_hbm,
                idx_v, ones_v, zrow_v, acc_sh):
    c = lax.axis_index("c")
    s = lax.axis_index("s")
    pltpu.sync_copy(zeros_hbm, zrow_v)
    pltpu.sync_copy(zrow_v, acc_sh.at[pl.ds(s * STRIPE, STRIPE)])
    pltpu.sync_copy(ones_hbm, ones_v)
    plsc.subcore_barrier()
    pltpu.sync_copy(edges_hbm.at[c, s], idx_v)

    def body(j, carry):
        pltpu.sync_copy(ones_v, acc_sh.at[idx_v.at[j]], add=True)
        return carry

    lax.fori_loop(0, _DEG_RPT, body, 0)
    plsc.subcore_barrier()
    pltpu.sync_copy(acc_sh.at[pl.ds(s * STRIPE, STRIPE)],
                    out_hbm.at[c, pl.ds(s * STRIPE, STRIPE)])


_CONV_RPT = (ER // NC) // NS
_NBUF = 5


def _make_conv(F):
    @functools.partial(
        pl.kernel,
        out_type=jax.ShapeDtypeStruct((2, N_PAD, F), jnp.float32),
        mesh=_MESH,
        compiler_params=_SC_PARAMS,
        scratch_types=[
            pltpu.VMEM((_CONV_RPT, CH), jnp.int32),
            pltpu.VMEM((_CONV_RPT, CH), jnp.int32),
            pltpu.VMEM((_NBUF, CH, F), jnp.float32),
            pltpu.VMEM((STRIPE, F), jnp.float32),
            pltpu.VMEM_SHARED((N_PAD, F), jnp.float32),
            pltpu.SemaphoreType.DMA((_NBUF,)),
        ],
    )
    def conv(feat_hbm, edges_hbm, zeros_hbm, out_hbm,
             src_v, dst_v, rows_s, zrow_v, acc_sh, sems):
        c = lax.axis_index("c")
        s = lax.axis_index("s")
        pltpu.sync_copy(zeros_hbm, zrow_v)
        pltpu.sync_copy(zrow_v, acc_sh.at[pl.ds(s * STRIPE, STRIPE)])
        plsc.subcore_barrier()
        w = c * NS + s
        pltpu.sync_copy(edges_hbm.at[0, w], src_v)
        pltpu.sync_copy(edges_hbm.at[1, w], dst_v)

        def start(j, b):
            pltpu.async_copy(feat_hbm.at[src_v.at[j]], rows_s.at[b],
                             sems.at[b])

        def wait(b):
            pltpu.make_async_copy(feat_hbm.at[pl.ds(0, CH)], rows_s.at[b],
                                  sems.at[b]).wait()

        for b in range(_NBUF - 1):
            start(b, b)

        def body(t, carry):
            j0 = t * _NBUF
            for b in range(_NBUF):
                j = j0 + b
                pf = jnp.minimum(j + _NBUF - 1, _CONV_RPT - 1)
                start(pf, (b + _NBUF - 1) % _NBUF)
                wait(b)
                pltpu.sync_copy(rows_s.at[b], acc_sh.at[dst_v.at[j]],
                                add=True)
            return carry

        lax.fori_loop(0, _CONV_RPT // _NBUF, body, 0)
        for b in range(_NBUF - 1):
            wait(b)
        plsc.subcore_barrier()
        pltpu.sync_copy(acc_sh.at[pl.ds(s * STRIPE, STRIPE)],
                        out_hbm.at[c, pl.ds(s * STRIPE, STRIPE)])

    return conv


_conv32 = _make_conv(EMB)
_conv64 = _make_conv(2 * EMB)

_DEC_RPT = ER // NS
_DEC_EPT = _DEC_RPT * CH


@functools.partial(
    pl.kernel,
    out_type=jax.ShapeDtypeStruct((2, NS, _DEC_RPT, CH), jnp.float32),
    mesh=_MESH,
    compiler_params=_SC_PARAMS,
    scratch_types=[
        pltpu.VMEM((_DEC_RPT, CH), jnp.int32),
        pltpu.VMEM((_DEC_RPT, CH), jnp.int32),
        pltpu.VMEM((_NBUF, CH, EMB), jnp.float32),
        pltpu.VMEM((_NBUF, CH, EMB), jnp.float32),
        pltpu.VMEM((_DEC_RPT, CH), jnp.float32),
        pltpu.SemaphoreType.DMA((_NBUF,)),
        pltpu.SemaphoreType.DMA((_NBUF,)),
    ],
)
def _dec_kernel(z_hbm, dec_hbm, out_hbm,
                u_v, v_v, zu_s, zv_s, res_v, sem_u, sem_v):
    c = lax.axis_index("c")
    s = lax.axis_index("s")
    pltpu.sync_copy(dec_hbm.at[c, 0, s], u_v)
    pltpu.sync_copy(dec_hbm.at[c, 1, s], v_v)
    lanes = lax.iota(jnp.int32, 16)

    def start(j, b):
        pltpu.async_copy(z_hbm.at[u_v.at[j]], zu_s.at[b], sem_u.at[b])
        pltpu.async_copy(z_hbm.at[v_v.at[j]], zv_s.at[b], sem_v.at[b])

    def wait(b):
        pltpu.make_async_copy(z_hbm.at[pl.ds(0, CH)], zu_s.at[b],
                              sem_u.at[b]).wait()
        pltpu.make_async_copy(z_hbm.at[pl.ds(0, CH)], zv_s.at[b],
                              sem_v.at[b]).wait()

    for b in range(_NBUF - 1):
        start(b, b)

    def body(t, carry):
        j0 = t * _NBUF
        for b in range(_NBUF):
            j = j0 + b
            pf = jnp.minimum(j + _NBUF - 1, _DEC_RPT - 1)
            start(pf, (b + _NBUF - 1) % _NBUF)
            wait(b)
            for g in range(CH // 16):
                rows = lanes + (g * 16)
                acc = jnp.zeros((16,), jnp.float32)
                for k in range(EMB):
                    kk = jnp.full((16,), k, jnp.int32)
                    acc = acc + (plsc.load_gather(zu_s.at[b], [rows, kk]) *
                                 plsc.load_gather(zv_s.at[b], [rows, kk]))
                res_v[j, pl.ds(g * 16, 16)] = acc
        return carry

    lax.fori_loop(0, _DEC_RPT // _NBUF, body, 0)
    for b in range(_NBUF - 1):
        wait(b)
    pltpu.sync_copy(res_v, out_hbm.at[c, s])


def _rs(deg2d):
    return lax.rsqrt(jnp.maximum(jnp.sum(deg2d, axis=-1, keepdims=True), 1.0))


def _enc1_body(x_ref, w_ref, d_ref, f1_ref):
    rs_out = _rs(d_ref[0])
    y1 = jnp.dot(x_ref[...], w_ref[...], preferred_element_type=jnp.float32,
                 precision=lax.Precision.HIGHEST)
    f1_ref[...] = y1 * rs_out


def _enc2_body(agg_ref, d_ref, w2_ref, b1_ref, f2_ref):
    agg = agg_ref[0] + agg_ref[1]
    h1 = jnp.maximum(agg * _rs(d_ref[1]) + b1_ref[...], 0.0)
    y2 = jnp.dot(h1, w2_ref[...], preferred_element_type=jnp.float32,
                 precision=lax.Precision.HIGHEST)
    f2_ref[...] = y2 * _rs(d_ref[0])


def _fin_body(agg_ref, d_ref, b2_ref, eps_ref, mu_ref, sig_ref, z_ref):
    agg = agg_ref[0] + agg_ref[1]
    h2 = agg * _rs(d_ref[1]) + b2_ref[...]
    mu = h2[:, :EMB]
    sig = jnp.exp(h2[:, EMB:] * 0.5)
    mu_ref[...] = mu
    sig_ref[...] = sig
    z_ref[...] = mu + sig * eps_ref[...]


_enc1 = pl.pallas_call(
    _enc1_body, out_shape=jax.ShapeDtypeStruct((N_PAD, EMB), jnp.float32))
_enc2 = pl.pallas_call(
    _enc2_body, out_shape=jax.ShapeDtypeStruct((N_PAD, 2 * EMB), jnp.float32))
_fin = pl.pallas_call(
    _fin_body, out_shape=[jax.ShapeDtypeStruct((N_PAD, EMB), jnp.float32),
                          jax.ShapeDtypeStruct((N_PAD, EMB), jnp.float32),
                          jax.ShapeDtypeStruct((N_PAD, EMB), jnp.float32)])


def kernel(x, W1, b1, W2, b2, eps, edge_index, pos_edge_index, neg_edge_index):
    x_p = jnp.pad(x, ((0, N_PAD - N), (0, 0)))
    eps_p = jnp.pad(eps, ((0, N_PAD - N), (0, 0)))
    edges_deg = edge_index.reshape(2, NS, _DEG_RPT, CH)
    edges_conv = edge_index.reshape(2, NC * NS, _CONV_RPT, CH)
    dec_r = jnp.stack([pos_edge_index.reshape(2, NS, _DEC_RPT, CH),
                       neg_edge_index.reshape(2, NS, _DEC_RPT, CH)])
    ones8 = jnp.zeros((CH, 8), jnp.float32).at[:, 0].set(1.0)
    z8 = jnp.zeros((STRIPE, 8), jnp.float32)
    z32 = jnp.zeros((STRIPE, EMB), jnp.float32)
    z64 = jnp.zeros((STRIPE, 2 * EMB), jnp.float32)

    deg = _deg_kernel(edges_deg, ones8, z8)
    f1 = _enc1(x_p, W1, deg)
    agg1 = _conv32(f1, edges_conv, z32)
    f2 = _enc2(agg1, deg, W2, b1.reshape(1, EMB))
    agg2 = _conv64(f2, edges_conv, z64)
    mu_p, sig_p, z_p = _fin(agg2, deg, b2.reshape(1, 2 * EMB), eps_p)
    scores = _dec_kernel(z_p, dec_r).reshape(2, E)
    return scores[0], scores[1], mu_p[:N], sig_p[:N]


# speedup vs baseline: 14.5527x; 2.1932x over previous
import functools

import jax
import jax.numpy as jnp
from jax import lax
from jax.experimental import pallas as pl
from jax.experimental.pallas import tpu as pltpu
from jax.experimental.pallas import tpu_sc as plsc

N = 10000
E = 320000
D_IN = 128
EMB = 32

NC = 2
NS = 16
N_PAD = 10240
STRIPE = N_PAD // NS
CH = 80
ER = E // CH

_MESH = plsc.VectorSubcoreMesh(
    core_axis_name="c", subcore_axis_name="s", num_cores=NC, num_subcores=NS)
_SC_PARAMS = pltpu.CompilerParams(
    use_tc_tiling_on_sc=False, needs_layout_passes=False)


_DEG_RPT = ER // NS


@functools.partial(
    pl.kernel,
    out_type=jax.ShapeDtypeStruct((2, N_PAD, 8), jnp.float32),
    mesh=_MESH,
    compiler_params=_SC_PARAMS,
    scratch_types=[
        pltpu.VMEM((_DEG_RPT, CH), jnp.int32),
        pltpu.VMEM((CH, 8), jnp.float32),
        pltpu.VMEM((STRIPE, 8), jnp.float32),
        pltpu.VMEM_SHARED((N_PAD, 8), jnp.float32),
    ],
)
def _deg_kernel(edges_hbm, ones_hbm, zeros_hbm, out_hbm,
                idx_v, ones_v, zrow_v, acc_sh):
    c = lax.axis_index("c")
    s = lax.axis_index("s")
    pltpu.sync_copy(zeros_hbm, zrow_v)
    pltpu.sync_copy(zrow_v, acc_sh.at[pl.ds(s * STRIPE, STRIPE)])
    pltpu.sync_copy(ones_hbm, ones_v)
    plsc.subcore_barrier()
    pltpu.sync_copy(edges_hbm.at[c, s], idx_v)

    def body(j, carry):
        pltpu.sync_copy(ones_v, acc_sh.at[idx_v.at[j]], add=True)
        return carry

    lax.fori_loop(0, _DEG_RPT, body, 0)
    plsc.subcore_barrier()
    pltpu.sync_copy(acc_sh.at[pl.ds(s * STRIPE, STRIPE)],
                    out_hbm.at[c, pl.ds(s * STRIPE, STRIPE)])


_CONV_RPT = (ER // NC) // NS
_NBUF = 5


def _make_conv(F):
    @functools.partial(
        pl.kernel,
        out_type=jax.ShapeDtypeStruct((2, N_PAD, F), jnp.float32),
        mesh=_MESH,
        compiler_params=_SC_PARAMS,
        scratch_types=[
            pltpu.VMEM((_CONV_RPT, CH), jnp.int32),
            pltpu.VMEM((_CONV_RPT, CH), jnp.int32),
            pltpu.VMEM((_NBUF, CH, F), jnp.float32),
            pltpu.VMEM((STRIPE, F), jnp.float32),
            pltpu.VMEM_SHARED((N_PAD, F), jnp.float32),
            pltpu.SemaphoreType.DMA((_NBUF,)),
        ],
    )
    def conv(feat_hbm, edges_hbm, zeros_hbm, out_hbm,
             src_v, dst_v, rows_s, zrow_v, acc_sh, sems):
        c = lax.axis_index("c")
        s = lax.axis_index("s")
        pltpu.sync_copy(zeros_hbm, zrow_v)
        pltpu.sync_copy(zrow_v, acc_sh.at[pl.ds(s * STRIPE, STRIPE)])
        plsc.subcore_barrier()
        w = c * NS + s
        pltpu.sync_copy(edges_hbm.at[0, w], src_v)
        pltpu.sync_copy(edges_hbm.at[1, w], dst_v)

        def start(j, b):
            pltpu.async_copy(feat_hbm.at[src_v.at[j]], rows_s.at[b],
                             sems.at[b])

        def wait(b):
            pltpu.make_async_copy(feat_hbm.at[pl.ds(0, CH)], rows_s.at[b],
                                  sems.at[b]).wait()

        for b in range(_NBUF - 1):
            start(b, b)

        def body(t, carry):
            j0 = t * _NBUF
            for b in range(_NBUF):
                j = j0 + b
                pf = jnp.minimum(j + _NBUF - 1, _CONV_RPT - 1)
                start(pf, (b + _NBUF - 1) % _NBUF)
                wait(b)
                pltpu.sync_copy(rows_s.at[b], acc_sh.at[dst_v.at[j]],
                                add=True)
            return carry

        lax.fori_loop(0, _CONV_RPT // _NBUF, body, 0)
        for b in range(_NBUF - 1):
            wait(b)
        plsc.subcore_barrier()
        pltpu.sync_copy(acc_sh.at[pl.ds(s * STRIPE, STRIPE)],
                        out_hbm.at[c, pl.ds(s * STRIPE, STRIPE)])

    return conv


_conv32 = _make_conv(EMB)
_conv64 = _make_conv(2 * EMB)

_DEC_RPT = ER // NS
_DEC_EPT = _DEC_RPT * CH


_KSLAB = 4


@functools.partial(
    pl.kernel,
    out_type=jax.ShapeDtypeStruct((2, NS, _DEC_RPT, CH), jnp.float32),
    mesh=_MESH,
    compiler_params=_SC_PARAMS,
    scratch_types=[
        pltpu.VMEM((_DEC_RPT, CH), jnp.int32),
        pltpu.VMEM((_DEC_RPT, CH), jnp.int32),
        pltpu.VMEM((_KSLAB, N_PAD), jnp.float32),
        pltpu.VMEM((_DEC_RPT, CH), jnp.float32),
    ],
)
def _dec_kernel(zt_hbm, dec_hbm, out_hbm, u_v, v_v, slab_v, res_v):
    c = lax.axis_index("c")
    s = lax.axis_index("s")
    pltpu.sync_copy(dec_hbm.at[c, 0, s], u_v)
    pltpu.sync_copy(dec_hbm.at[c, 1, s], v_v)

    for p in range(EMB // _KSLAB):
        pltpu.sync_copy(zt_hbm.at[pl.ds(p * _KSLAB, _KSLAB)], slab_v)

        def body(j, carry, first=(p == 0)):
            for g in range(CH // 16):
                una = u_v[j, pl.ds(g * 16, 16)]
                vna = v_v[j, pl.ds(g * 16, 16)]
                if first:
                    acc = jnp.zeros((16,), jnp.float32)
                else:
                    acc = res_v[j, pl.ds(g * 16, 16)]
                for kl in range(_KSLAB):
                    kk = jnp.full((16,), kl, jnp.int32)
                    acc = acc + (plsc.load_gather(slab_v, [kk, una]) *
                                 plsc.load_gather(slab_v, [kk, vna]))
                res_v[j, pl.ds(g * 16, 16)] = acc
            return carry

        lax.fori_loop(0, _DEC_RPT, body, 0)
    pltpu.sync_copy(res_v, out_hbm.at[c, s])


def _rs(deg2d):
    return lax.rsqrt(jnp.maximum(jnp.sum(deg2d, axis=-1, keepdims=True), 1.0))


def _enc1_body(x_ref, w_ref, d_ref, f1_ref):
    rs_out = _rs(d_ref[0])
    y1 = jnp.dot(x_ref[...], w_ref[...], preferred_element_type=jnp.float32,
                 precision=lax.Precision.HIGHEST)
    f1_ref[...] = y1 * rs_out


def _enc2_body(agg_ref, d_ref, w2_ref, b1_ref, f2_ref):
    agg = agg_ref[0] + agg_ref[1]
    h1 = jnp.maximum(agg * _rs(d_ref[1]) + b1_ref[...], 0.0)
    y2 = jnp.dot(h1, w2_ref[...], preferred_element_type=jnp.float32,
                 precision=lax.Precision.HIGHEST)
    f2_ref[...] = y2 * _rs(d_ref[0])


def _fin_body(agg_ref, d_ref, b2_ref, eps_ref, mu_ref, sig_ref, zt_ref):
    agg = agg_ref[0] + agg_ref[1]
    h2 = agg * _rs(d_ref[1]) + b2_ref[...]
    mu = h2[:, :EMB]
    sig = jnp.exp(h2[:, EMB:] * 0.5)
    mu_ref[...] = mu
    sig_ref[...] = sig
    zt_ref[...] = jnp.transpose(mu + sig * eps_ref[...])


_enc1 = pl.pallas_call(
    _enc1_body, out_shape=jax.ShapeDtypeStruct((N_PAD, EMB), jnp.float32))
_enc2 = pl.pallas_call(
    _enc2_body, out_shape=jax.ShapeDtypeStruct((N_PAD, 2 * EMB), jnp.float32))
_fin = pl.pallas_call(
    _fin_body, out_shape=[jax.ShapeDtypeStruct((N_PAD, EMB), jnp.float32),
                          jax.ShapeDtypeStruct((N_PAD, EMB), jnp.float32),
                          jax.ShapeDtypeStruct((EMB, N_PAD), jnp.float32)])


def kernel(x, W1, b1, W2, b2, eps, edge_index, pos_edge_index, neg_edge_index):
    x_p = jnp.pad(x, ((0, N_PAD - N), (0, 0)))
    eps_p = jnp.pad(eps, ((0, N_PAD - N), (0, 0)))
    edges_deg = edge_index.reshape(2, NS, _DEG_RPT, CH)
    edges_conv = edge_index.reshape(2, NC * NS, _CONV_RPT, CH)
    dec_r = jnp.stack([pos_edge_index.reshape(2, NS, _DEC_RPT, CH),
                       neg_edge_index.reshape(2, NS, _DEC_RPT, CH)])
    ones8 = jnp.zeros((CH, 8), jnp.float32).at[:, 0].set(1.0)
    z8 = jnp.zeros((STRIPE, 8), jnp.float32)
    z32 = jnp.zeros((STRIPE, EMB), jnp.float32)
    z64 = jnp.zeros((STRIPE, 2 * EMB), jnp.float32)

    deg = _deg_kernel(edges_deg, ones8, z8)
    f1 = _enc1(x_p, W1, deg)
    agg1 = _conv32(f1, edges_conv, z32)
    f2 = _enc2(agg1, deg, W2, b1.reshape(1, EMB))
    agg2 = _conv64(f2, edges_conv, z64)
    mu_p, sig_p, zt_p = _fin(agg2, deg, b2.reshape(1, 2 * EMB), eps_p)
    scores = _dec_kernel(zt_p, dec_r).reshape(2, E)
    return scores[0], scores[1], mu_p[:N], sig_p[:N]
